# baseline (speedup 1.0000x reference)
import jax
import jax.numpy as jnp
from jax.experimental import pallas as pl
from jax.experimental.pallas import tpu as pltpu

_N = 2048
_D = 128
_NUM_BINS = 10
_NUM_EDGES = _NUM_BINS + 1
_NUM_CLASSES = 64
_TI = 512
_GRID = _N // _TI


def _fastap_body(emb_full_ref, lab_full_ref, lab_tile_ref,
                 out_ref, embn_ref, ud_ref, y_ref, acc_ref):
    i = pl.program_id(0)

    @pl.when(i == 0)
    def _prep():
        emb = emb_full_ref[...]
        nrm = jnp.sqrt(jnp.sum(emb * emb, axis=1, keepdims=True))
        embn = emb / jnp.maximum(nrm, 1e-12)
        embn_ref[...] = embn.astype(jnp.bfloat16)
        g_d = jnp.sum(embn * embn, axis=1, keepdims=True)
        ud_ref[...] = jnp.maximum(5.0 - 5.0 * g_d, 0.0)
        cc_n = jax.lax.broadcasted_iota(jnp.int32, (_N, 2 * _NUM_CLASSES), 1)
        lab_n = lab_full_ref[...]
        y_ref[...] = (((lab_n == cc_n) & (cc_n < _NUM_CLASSES)) |
                      (cc_n == _NUM_CLASSES)).astype(jnp.bfloat16)
        acc_ref[0] = 0.0
        acc_ref[1] = 0.0

    etn = embn_ref[pl.ds(i * _TI, _TI), :]
    g = jax.lax.dot_general(etn, embn_ref[...], (((1,), (1,)), ((), ())),
                            preferred_element_type=jnp.float32)
    u = jnp.maximum(5.0 - 5.0 * g, 0.0).astype(jnp.bfloat16)
    u_d = ud_ref[pl.ds(i * _TI, _TI), :][:, 0]
    y = y_ref[...]

    cc_t = jax.lax.broadcasted_iota(jnp.int32, (_TI, 2 * _NUM_CLASSES), 1)
    lab_t = lab_tile_ref[...]
    yt = ((lab_t == cc_t) & (cc_t < _NUM_CLASSES)).astype(jnp.float32)

    one = jnp.bfloat16(1.0)
    zero = jnp.bfloat16(0.0)

    hp_cols = []
    ha_cols = []
    for k in range(_NUM_EDGES):
        t = jnp.clip(jnp.bfloat16(k + 1.0) - u, zero, one)
        t_d = jnp.clip((k + 1.0) - u_d, 0.0, 1.0)
        h = jax.lax.dot_general(t, y, (((1,), (0,)), ((), ())),
                                preferred_element_type=jnp.float32)
        hp_cols.append(jnp.sum(h * yt, axis=1) - t_d)
        ha_cols.append(h[:, _NUM_CLASSES] - t_d)
    h_pos_c = jnp.stack(hp_cols, axis=1)
    h_all_c = jnp.stack(ha_cols, axis=1)

    pos_hist = h_pos_c - jnp.concatenate(
        [jnp.zeros((_TI, 1), jnp.float32), h_pos_c[:, :_NUM_BINS]], axis=1)

    hp_prod = pos_hist * h_pos_c
    safe_h = (hp_prod > 0.0) & (h_all_c > 0.0)
    terms = jnp.where(safe_h, hp_prod / jnp.where(safe_h, h_all_c, 1.0), 0.0)
    fast_ap = jnp.sum(terms, axis=1)

    n_pos = h_pos_c[:, _NUM_BINS]
    safe_n = n_pos > 0.0
    fap = jnp.where(safe_n, fast_ap / jnp.where(safe_n, n_pos, 1.0), 0.0)
    acc_ref[0] += jnp.sum(jnp.where(safe_n, 1.0 - fap, 0.0))
    acc_ref[1] += jnp.sum(safe_n.astype(jnp.float32))

    @pl.when(i == _GRID - 1)
    def _fin():
        loss = acc_ref[0] / jnp.maximum(acc_ref[1], 1.0)
        out_ref[...] = jnp.full((1, 1), loss, jnp.float32)


def kernel(embeddings, labels):
    lab2d = labels.reshape(_N, 1)
    out = pl.pallas_call(
        _fastap_body,
        grid=(_GRID,),
        in_specs=[
            pl.BlockSpec((_N, _D), lambda i: (0, 0)),
            pl.BlockSpec((_N, 1), lambda i: (0, 0)),
            pl.BlockSpec((_TI, 1), lambda i: (i, 0)),
        ],
        out_specs=pl.BlockSpec((1, 1), lambda i: (0, 0)),
        out_shape=jax.ShapeDtypeStruct((1, 1), jnp.float32),
        scratch_shapes=[
            pltpu.VMEM((_N, _D), jnp.bfloat16),
            pltpu.VMEM((_N, 1), jnp.float32),
            pltpu.VMEM((_N, 2 * _NUM_CLASSES), jnp.bfloat16),
            pltpu.SMEM((2,), jnp.float32),
        ],
    )(embeddings, lab2d, lab2d)
    return out.reshape(())


# speedup vs baseline: 1.8677x; 1.8677x over previous
import jax
import jax.numpy as jnp
from jax.experimental import pallas as pl
from jax.experimental.pallas import tpu as pltpu

_N = 2048
_D = 128
_NUM_BINS = 10
_NUM_EDGES = _NUM_BINS + 1
_NUM_CLASSES = 64
_TI = 1024
_GRID = _N // _TI
_N_MXU_BINS = 2


def _row_tree_sum(x):
    n = 2048
    while n > 128:
        n //= 2
        x = x[:, :n] + x[:, n:2 * n]
    return jnp.sum(x.astype(jnp.float32), axis=1)


def _fastap_body(emb_full_ref, emb_tile_ref, lab_full_ref, lab_tile_ref,
                 lab_col_ref, out_ref, acc_ref):
    i = pl.program_id(0)

    emb = emb_full_ref[...]
    nrm = jnp.sqrt(jnp.sum(emb * emb, axis=1, keepdims=True))
    embn = emb / jnp.maximum(nrm, 1e-12)

    et = emb_tile_ref[...]
    nrm_t = jnp.sqrt(jnp.sum(et * et, axis=1, keepdims=True))
    etn = et / jnp.maximum(nrm_t, 1e-12)
    etn5 = 5.0 * etn

    g5 = jax.lax.dot_general(etn5.astype(jnp.bfloat16),
                             embn.astype(jnp.bfloat16),
                             (((1,), (1,)), ((), ())),
                             preferred_element_type=jnp.float32
                             ).astype(jnp.bfloat16)

    g5_d = 5.0 * jnp.sum(etn * etn, axis=1)

    cc_n = jax.lax.broadcasted_iota(jnp.int32, (_N, 2 * _NUM_CLASSES), 1)
    lab_n = lab_full_ref[...]
    y = (((lab_n == cc_n) & (cc_n < _NUM_CLASSES)) |
         (cc_n == _NUM_CLASSES)).astype(jnp.bfloat16)
    y_cls = ((lab_n == cc_n) &
             (cc_n < _NUM_CLASSES)).astype(jnp.bfloat16)

    cc_t = jax.lax.broadcasted_iota(jnp.int32, (_TI, 2 * _NUM_CLASSES), 1)
    lab_t = lab_tile_ref[...]
    yt_b = ((lab_t == cc_t) & (cc_t < _NUM_CLASSES)).astype(jnp.bfloat16)
    yt = yt_b.astype(jnp.float32)

    samef = (lab_t == lab_col_ref[...]).astype(jnp.bfloat16)

    one = jnp.bfloat16(1.0)
    zero = jnp.bfloat16(0.0)

    hp_cols = []
    ha_cols = []
    for k in range(_NUM_EDGES):
        t = jnp.clip(g5 + jnp.bfloat16(k - 4.0), zero, one)
        t_d = jnp.clip(g5_d + (k - 4.0), 0.0, 1.0)
        if k < _N_MXU_BINS:
            h = jax.lax.dot_general(t, y, (((1,), (0,)), ((), ())),
                                    preferred_element_type=jnp.float32)
            hp_cols.append(jnp.sum(h * yt, axis=1) - t_d)
            ha_cols.append(h[:, _NUM_CLASSES] - t_d)
        else:
            hp_cols.append(_row_tree_sum(t * samef) - t_d)
            ha_cols.append(_row_tree_sum(t) - t_d)
    h_pos_c = jnp.stack(hp_cols, axis=1)
    h_all_c = jnp.stack(ha_cols, axis=1)

    pos_hist = h_pos_c - jnp.concatenate(
        [jnp.zeros((_TI, 1), jnp.float32), h_pos_c[:, :_NUM_BINS]], axis=1)

    hp_prod = pos_hist * h_pos_c
    safe_h = (hp_prod > 0.0) & (h_all_c > 0.0)
    terms = jnp.where(safe_h, hp_prod / jnp.where(safe_h, h_all_c, 1.0), 0.0)
    fast_ap = jnp.sum(terms, axis=1)

    n_pos = h_pos_c[:, _NUM_BINS]
    safe_n = n_pos > 0.0
    fap = jnp.where(safe_n, fast_ap / jnp.where(safe_n, n_pos, 1.0), 0.0)
    num_t = jnp.sum(jnp.where(safe_n, 1.0 - fap, 0.0))
    cnt_t = jnp.sum(safe_n.astype(jnp.float32))

    @pl.when(i == 0)
    def _init():
        acc_ref[0] = 0.0
        acc_ref[1] = 0.0

    acc_ref[0] += num_t
    acc_ref[1] += cnt_t

    @pl.when(i == _GRID - 1)
    def _fin():
        loss = acc_ref[0] / jnp.maximum(acc_ref[1], 1.0)
        out_ref[...] = jnp.full((1, 1), loss, jnp.float32)


def kernel(embeddings, labels):
    lab2d = labels.reshape(_N, 1)
    out = pl.pallas_call(
        _fastap_body,
        grid=(_GRID,),
        in_specs=[
            pl.BlockSpec((_N, _D), lambda i: (0, 0)),
            pl.BlockSpec((_TI, _D), lambda i: (i, 0)),
            pl.BlockSpec((_N, 1), lambda i: (0, 0)),
            pl.BlockSpec((_TI, 1), lambda i: (i, 0)),
            pl.BlockSpec((1, _N), lambda i: (0, 0)),
        ],
        out_specs=pl.BlockSpec((1, 1), lambda i: (0, 0)),
        out_shape=jax.ShapeDtypeStruct((1, 1), jnp.float32),
        scratch_shapes=[pltpu.SMEM((2,), jnp.float32)],
    )(embeddings, embeddings, lab2d, lab2d, labels.reshape(1, _N))
    return out.reshape(())


# speedup vs baseline: 1.8908x; 1.0124x over previous
import jax
import jax.numpy as jnp
from jax.experimental import pallas as pl
from jax.experimental.pallas import tpu as pltpu

_N = 2048
_D = 128
_NUM_BINS = 10
_NUM_EDGES = _NUM_BINS + 1
_NUM_CLASSES = 64
_TI = 2048
_GRID = _N // _TI
_N_MXU_BINS = 2


def _row_tree_sum(x):
    n = 2048
    while n > 128:
        n //= 2
        x = x[:, :n] + x[:, n:2 * n]
    return jnp.sum(x.astype(jnp.float32), axis=1)


def _fastap_body(emb_full_ref, emb_tile_ref, lab_full_ref, lab_tile_ref,
                 lab_col_ref, out_ref, acc_ref):
    i = pl.program_id(0)

    emb = emb_full_ref[...]
    nrm = jnp.sqrt(jnp.sum(emb * emb, axis=1, keepdims=True))
    embn = emb / jnp.maximum(nrm, 1e-12)

    et = emb_tile_ref[...]
    nrm_t = jnp.sqrt(jnp.sum(et * et, axis=1, keepdims=True))
    etn = et / jnp.maximum(nrm_t, 1e-12)
    etn5 = 5.0 * etn

    g5 = jax.lax.dot_general(etn5.astype(jnp.bfloat16),
                             embn.astype(jnp.bfloat16),
                             (((1,), (1,)), ((), ())),
                             preferred_element_type=jnp.float32
                             ).astype(jnp.bfloat16)

    g5_d = 5.0 * jnp.sum(etn * etn, axis=1)

    cc_n = jax.lax.broadcasted_iota(jnp.int32, (_N, 2 * _NUM_CLASSES), 1)
    lab_n = lab_full_ref[...]
    y = (((lab_n == cc_n) & (cc_n < _NUM_CLASSES)) |
         (cc_n == _NUM_CLASSES)).astype(jnp.bfloat16)
    y_cls = ((lab_n == cc_n) &
             (cc_n < _NUM_CLASSES)).astype(jnp.bfloat16)

    cc_t = jax.lax.broadcasted_iota(jnp.int32, (_TI, 2 * _NUM_CLASSES), 1)
    lab_t = lab_tile_ref[...]
    yt_b = ((lab_t == cc_t) & (cc_t < _NUM_CLASSES)).astype(jnp.bfloat16)
    yt = yt_b.astype(jnp.float32)

    samef = (lab_t == lab_col_ref[...]).astype(jnp.bfloat16)

    one = jnp.bfloat16(1.0)
    zero = jnp.bfloat16(0.0)

    hp_cols = []
    ha_cols = []
    for k in range(_NUM_EDGES):
        t = jnp.clip(g5 + jnp.bfloat16(k - 4.0), zero, one)
        t_d = jnp.clip(g5_d + (k - 4.0), 0.0, 1.0)
        if k < _N_MXU_BINS:
            h = jax.lax.dot_general(t, y, (((1,), (0,)), ((), ())),
                                    preferred_element_type=jnp.float32)
            hp_cols.append(jnp.sum(h * yt, axis=1) - t_d)
            ha_cols.append(h[:, _NUM_CLASSES] - t_d)
        else:
            hp_cols.append(_row_tree_sum(t * samef) - t_d)
            ha_cols.append(_row_tree_sum(t) - t_d)
    h_pos_c = jnp.stack(hp_cols, axis=1)
    h_all_c = jnp.stack(ha_cols, axis=1)

    pos_hist = h_pos_c - jnp.concatenate(
        [jnp.zeros((_TI, 1), jnp.float32), h_pos_c[:, :_NUM_BINS]], axis=1)

    hp_prod = pos_hist * h_pos_c
    safe_h = (hp_prod > 0.0) & (h_all_c > 0.0)
    terms = jnp.where(safe_h, hp_prod / jnp.where(safe_h, h_all_c, 1.0), 0.0)
    fast_ap = jnp.sum(terms, axis=1)

    n_pos = h_pos_c[:, _NUM_BINS]
    safe_n = n_pos > 0.0
    fap = jnp.where(safe_n, fast_ap / jnp.where(safe_n, n_pos, 1.0), 0.0)
    num_t = jnp.sum(jnp.where(safe_n, 1.0 - fap, 0.0))
    cnt_t = jnp.sum(safe_n.astype(jnp.float32))

    @pl.when(i == 0)
    def _init():
        acc_ref[0] = 0.0
        acc_ref[1] = 0.0

    acc_ref[0] += num_t
    acc_ref[1] += cnt_t

    @pl.when(i == _GRID - 1)
    def _fin():
        loss = acc_ref[0] / jnp.maximum(acc_ref[1], 1.0)
        out_ref[...] = jnp.full((1, 1), loss, jnp.float32)


def kernel(embeddings, labels):
    lab2d = labels.reshape(_N, 1)
    out = pl.pallas_call(
        _fastap_body,
        grid=(_GRID,),
        in_specs=[
            pl.BlockSpec((_N, _D), lambda i: (0, 0)),
            pl.BlockSpec((_TI, _D), lambda i: (i, 0)),
            pl.BlockSpec((_N, 1), lambda i: (0, 0)),
            pl.BlockSpec((_TI, 1), lambda i: (i, 0)),
            pl.BlockSpec((1, _N), lambda i: (0, 0)),
        ],
        out_specs=pl.BlockSpec((1, 1), lambda i: (0, 0)),
        out_shape=jax.ShapeDtypeStruct((1, 1), jnp.float32),
        scratch_shapes=[pltpu.SMEM((2,), jnp.float32)],
    )(embeddings, embeddings, lab2d, lab2d, labels.reshape(1, _N))
    return out.reshape(())
